# baseline (speedup 1.0000x reference)
import jax
import jax.numpy as jnp
from jax.experimental import pallas as pl
from jax.experimental.pallas import tpu as pltpu

N, D, E, H, A = 32768, 768, 8, 256, 32
VF_H1, VF_H2 = 256, 128
TOK = 512


def _moe_block_kernel(x_ref, wg_ref, bg_ref, w1_ref, b1_ref, w2_ref, b2_ref,
                      wv1_ref, bv1_ref, wv2_ref, bv2_ref, pi_ref, vf_ref):
    x = x_ref[...]

    logits = jax.lax.dot_general(
        x, wg_ref[...], (((1,), (0,)), ((), ())),
        preferred_element_type=jnp.float32) + bg_ref[...]
    lane = jax.lax.broadcasted_iota(jnp.int32, logits.shape, 1)
    m1 = jnp.max(logits, axis=-1, keepdims=True)
    i1 = jnp.min(jnp.where(logits == m1, lane, E), axis=-1, keepdims=True)
    l2 = jnp.where(lane == i1, -jnp.inf, logits)
    m2 = jnp.max(l2, axis=-1, keepdims=True)
    i2 = jnp.min(jnp.where(l2 == m2, lane, E), axis=-1, keepdims=True)
    g1 = jax.nn.sigmoid(m1 - m2)
    g2 = 1.0 - g1

    xb = x.astype(jnp.bfloat16)
    h = jax.lax.dot_general(
        xb, w1_ref[...], (((1,), (0,)), ((), ())),
        preferred_element_type=jnp.float32) + b1_ref[...]
    hb = h.astype(jnp.bfloat16)
    s = hb * jax.nn.sigmoid(hb)
    o = jax.lax.dot_general(
        s, w2_ref[...], (((1,), (0,)), ((), ())),
        preferred_element_type=jnp.float32)

    elane = jax.lax.broadcasted_iota(jnp.int32, o.shape, 1) // A
    wmap = (g1 * (elane == i1).astype(jnp.float32)
            + g2 * (elane == i2).astype(jnp.float32))
    ow = o * wmap
    r = ow[:, :128] + ow[:, 128:]
    r = r[:, :64] + r[:, 64:]
    pi = r[:, :A] + r[:, A:]
    w8 = (g1 * (lane == i1).astype(jnp.float32)
          + g2 * (lane == i2).astype(jnp.float32))
    pi = pi + jax.lax.dot_general(
        w8, b2_ref[...], (((1,), (0,)), ((), ())),
        preferred_element_type=jnp.float32)
    pi_ref[...] = pi

    v = jax.lax.dot_general(
        xb, wv1_ref[...], (((1,), (0,)), ((), ())),
        preferred_element_type=jnp.float32) + bv1_ref[...]
    vb = v.astype(jnp.bfloat16)
    vs = vb * jax.nn.sigmoid(vb)
    vf = jax.lax.dot_general(
        vs, wv2_ref[...], (((1,), (0,)), ((), ())),
        preferred_element_type=jnp.float32) + bv2_ref[...]
    vf_ref[...] = vf * jax.nn.sigmoid(vf)


def kernel(features, Wg, bg, W1, b1, W2, b2, Wv1, bv1, Wv2, bv2):
    w1f = W1.transpose(1, 0, 2).reshape(D, E * H).astype(jnp.bfloat16)
    b1f = b1.reshape(1, E * H)
    eye = jnp.eye(E, dtype=jnp.float32)
    w2bd = jnp.einsum('ef,eha->ehfa', eye, W2).reshape(E * H, E * A)
    w2bd = w2bd.astype(jnp.bfloat16)
    wv1b = Wv1.astype(jnp.bfloat16)
    wv2b = Wv2.astype(jnp.bfloat16)

    grid = (N // TOK,)
    full = lambda *shape: pl.BlockSpec(shape, lambda i: (0,) * len(shape))
    pi, vf = pl.pallas_call(
        _moe_block_kernel,
        grid=grid,
        in_specs=[
            pl.BlockSpec((TOK, D), lambda i: (i, 0)),
            full(D, E),
            full(1, E),
            full(D, E * H),
            full(1, E * H),
            full(E * H, E * A),
            full(E, A),
            full(D, VF_H1),
            full(1, VF_H1),
            full(VF_H1, VF_H2),
            full(1, VF_H2),
        ],
        out_specs=[
            pl.BlockSpec((TOK, A), lambda i: (i, 0)),
            pl.BlockSpec((TOK, VF_H2), lambda i: (i, 0)),
        ],
        out_shape=[
            jax.ShapeDtypeStruct((N, A), jnp.float32),
            jax.ShapeDtypeStruct((N, VF_H2), jnp.float32),
        ],
        compiler_params=pltpu.CompilerParams(
            dimension_semantics=("arbitrary",)),
    )(features, Wg, bg.reshape(1, E), w1f, b1f, w2bd, b2,
      wv1b, bv1.reshape(1, VF_H1), wv2b, bv2.reshape(1, VF_H2))
    return (pi, vf)


# speedup vs baseline: 1.0519x; 1.0519x over previous
import jax
import jax.numpy as jnp
from jax.experimental import pallas as pl
from jax.experimental.pallas import tpu as pltpu

N, D, E, H, A = 32768, 768, 8, 256, 32
VF_H1, VF_H2 = 256, 128
TOK = 1024


def _moe_block_kernel(x_ref, wg_ref, bg_ref, w1_ref, b1_ref, w2_ref, b2_ref,
                      wv1_ref, bv1_ref, wv2_ref, bv2_ref, pi_ref, vf_ref):
    x = x_ref[...]

    logits = jax.lax.dot_general(
        x, wg_ref[...], (((1,), (0,)), ((), ())),
        preferred_element_type=jnp.float32) + bg_ref[...]
    lane = jax.lax.broadcasted_iota(jnp.int32, logits.shape, 1)
    m1 = jnp.max(logits, axis=-1, keepdims=True)
    i1 = jnp.min(jnp.where(logits == m1, lane, E), axis=-1, keepdims=True)
    l2 = jnp.where(lane == i1, -jnp.inf, logits)
    m2 = jnp.max(l2, axis=-1, keepdims=True)
    i2 = jnp.min(jnp.where(l2 == m2, lane, E), axis=-1, keepdims=True)
    g1 = jax.nn.sigmoid(m1 - m2)
    g2 = 1.0 - g1

    xb = x.astype(jnp.bfloat16)
    h = jax.lax.dot_general(
        xb, w1_ref[...], (((1,), (0,)), ((), ())),
        preferred_element_type=jnp.float32) + b1_ref[...]
    hb = h.astype(jnp.bfloat16)
    s = hb * jax.nn.sigmoid(hb)
    o = jax.lax.dot_general(
        s, w2_ref[...], (((1,), (0,)), ((), ())),
        preferred_element_type=jnp.float32)

    elane = jax.lax.broadcasted_iota(jnp.int32, o.shape, 1) // A
    wmap = (g1 * (elane == i1).astype(jnp.float32)
            + g2 * (elane == i2).astype(jnp.float32))
    ow = o * wmap
    r = ow[:, :128] + ow[:, 128:]
    r = r[:, :64] + r[:, 64:]
    pi = r[:, :A] + r[:, A:]
    w8 = (g1 * (lane == i1).astype(jnp.float32)
          + g2 * (lane == i2).astype(jnp.float32))
    pi = pi + jax.lax.dot_general(
        w8, b2_ref[...], (((1,), (0,)), ((), ())),
        preferred_element_type=jnp.float32)
    pi_ref[...] = pi

    v = jax.lax.dot_general(
        xb, wv1_ref[...], (((1,), (0,)), ((), ())),
        preferred_element_type=jnp.float32) + bv1_ref[...]
    vb = v.astype(jnp.bfloat16)
    vs = vb * jax.nn.sigmoid(vb)
    vf = jax.lax.dot_general(
        vs, wv2_ref[...], (((1,), (0,)), ((), ())),
        preferred_element_type=jnp.float32) + bv2_ref[...]
    vf_ref[...] = vf * jax.nn.sigmoid(vf)


def kernel(features, Wg, bg, W1, b1, W2, b2, Wv1, bv1, Wv2, bv2):
    w1f = W1.transpose(1, 0, 2).reshape(D, E * H).astype(jnp.bfloat16)
    b1f = b1.reshape(1, E * H)
    eye = jnp.eye(E, dtype=jnp.float32)
    w2bd = jnp.einsum('ef,eha->ehfa', eye, W2).reshape(E * H, E * A)
    w2bd = w2bd.astype(jnp.bfloat16)
    wv1b = Wv1.astype(jnp.bfloat16)
    wv2b = Wv2.astype(jnp.bfloat16)

    grid = (N // TOK,)
    full = lambda *shape: pl.BlockSpec(shape, lambda i: (0,) * len(shape))
    pi, vf = pl.pallas_call(
        _moe_block_kernel,
        grid=grid,
        in_specs=[
            pl.BlockSpec((TOK, D), lambda i: (i, 0)),
            full(D, E),
            full(1, E),
            full(D, E * H),
            full(1, E * H),
            full(E * H, E * A),
            full(E, A),
            full(D, VF_H1),
            full(1, VF_H1),
            full(VF_H1, VF_H2),
            full(1, VF_H2),
        ],
        out_specs=[
            pl.BlockSpec((TOK, A), lambda i: (i, 0)),
            pl.BlockSpec((TOK, VF_H2), lambda i: (i, 0)),
        ],
        out_shape=[
            jax.ShapeDtypeStruct((N, A), jnp.float32),
            jax.ShapeDtypeStruct((N, VF_H2), jnp.float32),
        ],
        compiler_params=pltpu.CompilerParams(
            dimension_semantics=("arbitrary",)),
    )(features, Wg, bg.reshape(1, E), w1f, b1f, w2bd, b2,
      wv1b, bv1.reshape(1, VF_H1), wv2b, bv2.reshape(1, VF_H2))
    return (pi, vf)
